# baseline (speedup 1.0000x reference)
import functools

import jax
import jax.numpy as jnp
from jax import lax
from jax.experimental import pallas as pl
from jax.experimental.pallas import tpu as pltpu
from jax.experimental.pallas import tpu_sc as plsc

NC = 2
NS = 16
NW = NC * NS
CHUNK = 128


def _make_deg_kernel(C, deg_slots):
    mesh = plsc.VectorSubcoreMesh(core_axis_name="c", subcore_axis_name="s")

    @functools.partial(
        pl.kernel,
        out_type=jax.ShapeDtypeStruct((NW * 2 * deg_slots,), jnp.float32),
        mesh=mesh,
        scratch_types=[
            pltpu.VMEM((C, CHUNK), jnp.int32),
            pltpu.VMEM((C, CHUNK), jnp.int32),
            pltpu.VMEM((deg_slots,), jnp.float32),
            pltpu.VMEM((deg_slots,), jnp.float32),
        ],
        compiler_params=pltpu.CompilerParams(needs_layout_passes=False),
    )
    def deg_kernel(src_hbm, dst_hbm, zeros_hbm, out_hbm,
                   src_v, dst_v, dego_v, degi_v):
        wid = lax.axis_index("s") * NC + lax.axis_index("c")
        pltpu.sync_copy(src_hbm.at[wid], src_v)
        pltpu.sync_copy(dst_hbm.at[wid], dst_v)
        pltpu.sync_copy(zeros_hbm, dego_v)
        pltpu.sync_copy(zeros_hbm, degi_v)
        ones = jnp.ones((16,), jnp.float32)

        @pl.loop(0, C)
        def _(r):
            for q in range(CHUNK // 16):
                s16 = src_v[r, pl.ds(q * 16, 16)]
                d16 = dst_v[r, pl.ds(q * 16, 16)]
                plsc.addupdate_scatter(dego_v, [s16], ones)
                plsc.addupdate_scatter(degi_v, [d16], ones)

        base = wid * 2 * deg_slots
        pltpu.sync_copy(dego_v, out_hbm.at[pl.ds(base, deg_slots)])
        pltpu.sync_copy(degi_v, out_hbm.at[pl.ds(base + deg_slots, deg_slots)])

    return deg_kernel


NBUF = 4


def _make_agg_cols_kernel(C, Dh, N, acc_rows):
    mesh = plsc.VectorSubcoreMesh(core_axis_name="c", subcore_axis_name="s")
    rows_pt = (N // NS) // 8 * 8
    rows_rem = N - NS * rows_pt
    zrows_pt = acc_rows // NS

    @functools.partial(
        pl.kernel,
        out_type=jax.ShapeDtypeStruct((NC, N, Dh), jnp.float32),
        mesh=mesh,
        scratch_types=[
            pltpu.VMEM((C, CHUNK), jnp.int32),
            pltpu.VMEM((C, CHUNK), jnp.int32),
            [pltpu.VMEM((CHUNK, Dh), jnp.float32) for _ in range(NBUF)],
            [pltpu.SemaphoreType.DMA for _ in range(NBUF)],
            pltpu.VMEM_SHARED((acc_rows, Dh), jnp.float32),
        ],
        compiler_params=pltpu.CompilerParams(use_tc_tiling_on_sc=False),
    )
    def agg_kernel(t2_hbm, src_hbm, dst_hbm, zeros_hbm, out_hbm,
                   src_v, dst_v, bufs, sems, acc):
        c = lax.axis_index("c")
        s = lax.axis_index("s")
        pltpu.sync_copy(src_hbm.at[c, s], src_v)
        pltpu.sync_copy(dst_hbm.at[s], dst_v)
        pltpu.sync_copy(zeros_hbm.at[pl.ds(s * zrows_pt, zrows_pt)],
                        acc.at[pl.ds(s * zrows_pt, zrows_pt)])
        plsc.subcore_barrier()

        for k in range(NBUF):
            pltpu.async_copy(t2_hbm.at[src_v.at[k]], bufs[k], sems[k])

        @pl.loop(0, C - NBUF, step=NBUF)
        def _(j):
            for k in range(NBUF):
                pltpu.make_async_copy(
                    t2_hbm.at[src_v.at[0]], bufs[k], sems[k]).wait()
                pltpu.sync_copy(bufs[k], acc.at[dst_v.at[j + k]], add=True)
                pltpu.async_copy(
                    t2_hbm.at[src_v.at[j + k + NBUF]], bufs[k], sems[k])

        for k in range(NBUF):
            pltpu.make_async_copy(
                t2_hbm.at[src_v.at[0]], bufs[k], sems[k]).wait()
            pltpu.sync_copy(bufs[k], acc.at[dst_v.at[C - NBUF + k]], add=True)

        plsc.subcore_barrier()
        pltpu.sync_copy(acc.at[pl.ds(s * rows_pt, rows_pt)],
                        out_hbm.at[c, pl.ds(s * rows_pt, rows_pt)])
        if rows_rem:
            @pl.when(s == 0)
            def _():
                pltpu.sync_copy(acc.at[pl.ds(NS * rows_pt, rows_rem)],
                                out_hbm.at[c, pl.ds(NS * rows_pt, rows_rem)])

    return agg_kernel


def _make_agg_kernel(C, D, N, acc_rows):
    mesh = plsc.VectorSubcoreMesh(core_axis_name="c", subcore_axis_name="s")
    rows_pt = (N // NS) // 8 * 8
    rows_rem = N - NS * rows_pt
    zrows_pt = acc_rows // NS

    @functools.partial(
        pl.kernel,
        out_type=jax.ShapeDtypeStruct((NC, N, D), jnp.float32),
        mesh=mesh,
        scratch_types=[
            pltpu.VMEM((C, CHUNK), jnp.int32),
            pltpu.VMEM((C, CHUNK), jnp.int32),
            [pltpu.VMEM((CHUNK, D), jnp.float32) for _ in range(NBUF)],
            [pltpu.SemaphoreType.DMA for _ in range(NBUF)],
            pltpu.VMEM_SHARED((acc_rows, D), jnp.float32),
        ],
        compiler_params=pltpu.CompilerParams(use_tc_tiling_on_sc=False),
    )
    def agg_kernel(t_hbm, src_hbm, dst_hbm, zeros_hbm, out_hbm,
                   src_v, dst_v, bufs, sems, acc):
        c = lax.axis_index("c")
        s = lax.axis_index("s")
        wid = s * NC + c
        pltpu.sync_copy(src_hbm.at[wid], src_v)
        pltpu.sync_copy(dst_hbm.at[wid], dst_v)
        pltpu.sync_copy(zeros_hbm.at[pl.ds(s * zrows_pt, zrows_pt)],
                        acc.at[pl.ds(s * zrows_pt, zrows_pt)])
        plsc.subcore_barrier()

        for k in range(NBUF):
            pltpu.async_copy(t_hbm.at[src_v.at[k]], bufs[k], sems[k])

        @pl.loop(0, C - NBUF, step=NBUF)
        def _(j):
            for k in range(NBUF):
                pltpu.make_async_copy(
                    t_hbm.at[src_v.at[0]], bufs[k], sems[k]).wait()
                pltpu.sync_copy(bufs[k], acc.at[dst_v.at[j + k]], add=True)
                pltpu.async_copy(
                    t_hbm.at[src_v.at[j + k + NBUF]], bufs[k], sems[k])

        for k in range(NBUF):
            pltpu.make_async_copy(
                t_hbm.at[src_v.at[0]], bufs[k], sems[k]).wait()
            pltpu.sync_copy(bufs[k], acc.at[dst_v.at[C - NBUF + k]], add=True)

        plsc.subcore_barrier()
        pltpu.sync_copy(acc.at[pl.ds(s * rows_pt, rows_pt)],
                        out_hbm.at[c, pl.ds(s * rows_pt, rows_pt)])
        if rows_rem:
            @pl.when(s == 0)
            def _():
                pltpu.sync_copy(acc.at[pl.ds(NS * rows_pt, rows_rem)],
                                out_hbm.at[c, pl.ds(NS * rows_pt, rows_rem)])

    return agg_kernel


def _norm_body(degp_ref, out_ref):
    d = jnp.sum(degp_ref[...], axis=0, keepdims=True) + 1.0
    out_ref[...] = lax.rsqrt(d)


def _l1_body(f_ref, no_ref, w_ref, out_ref):
    x = f_ref[...] * no_ref[...]
    out_ref[...] = jnp.dot(x, w_ref[...], preferred_element_type=jnp.float32)


def _mid_body(p_ref, t_ref, ni_ref, no_ref, b_ref, w_ref, out_ref):
    agg = jnp.concatenate([p_ref[0], p_ref[1]], axis=-1)
    x = (agg + t_ref[...]) * ni_ref[...] + b_ref[...]
    x = jnp.maximum(x, 0.0) * no_ref[...]
    out_ref[...] = jnp.dot(x, w_ref[...], preferred_element_type=jnp.float32)


def _fin_body(p_ref, t_ref, ni_ref, b_ref, out_ref):
    x = (p_ref[0] + p_ref[1] + t_ref[...]) * ni_ref[...] + b_ref[...]
    out_ref[...] = jax.nn.sigmoid(x) + 1e-8


def _norm_call(degp):
    nw, m = degp.shape
    return pl.pallas_call(
        _norm_body,
        out_shape=jax.ShapeDtypeStruct((1, m), jnp.float32),
    )(degp)


def _l1_call(features, n_out, W, rows):
    n, f = features.shape
    h = W.shape[1]
    grid = (n // rows,)
    return pl.pallas_call(
        _l1_body,
        grid=grid,
        in_specs=[
            pl.BlockSpec((rows, f), lambda i: (i, 0)),
            pl.BlockSpec((rows, 1), lambda i: (i, 0)),
            pl.BlockSpec((f, h), lambda i: (0, 0)),
        ],
        out_specs=pl.BlockSpec((rows, h), lambda i: (i, 0)),
        out_shape=jax.ShapeDtypeStruct((n, h), jnp.float32),
    )(features, n_out, W)


def _mid_call(p, t, n_in, n_out, b, W, rows):
    n, d = t.shape
    do = W.shape[1]
    grid = (n // rows,)
    return pl.pallas_call(
        _mid_body,
        grid=grid,
        in_specs=[
            pl.BlockSpec((NC, rows, d // 2), lambda i: (0, i, 0)),
            pl.BlockSpec((rows, d), lambda i: (i, 0)),
            pl.BlockSpec((rows, 1), lambda i: (i, 0)),
            pl.BlockSpec((rows, 1), lambda i: (i, 0)),
            pl.BlockSpec((1, d), lambda i: (0, 0)),
            pl.BlockSpec((d, do), lambda i: (0, 0)),
        ],
        out_specs=pl.BlockSpec((rows, do), lambda i: (i, 0)),
        out_shape=jax.ShapeDtypeStruct((n, do), jnp.float32),
    )(p, t, n_in, n_out, b, W)


def _fin_call(p, t, n_in, b, rows):
    n, d = t.shape
    grid = (n // rows,)
    return pl.pallas_call(
        _fin_body,
        grid=grid,
        in_specs=[
            pl.BlockSpec((NC, rows, d), lambda i: (0, i, 0)),
            pl.BlockSpec((rows, d), lambda i: (i, 0)),
            pl.BlockSpec((rows, 1), lambda i: (i, 0)),
            pl.BlockSpec((1, d), lambda i: (0, 0)),
        ],
        out_specs=pl.BlockSpec((rows, d), lambda i: (i, 0)),
        out_shape=jax.ShapeDtypeStruct((n, d), jnp.float32),
    )(p, t, n_in, b)


@jax.jit
def kernel(features, edge_index, W1, b1, W2, b2, W3, b3):
    N, F = features.shape
    E = edge_index.shape[1]
    H = W1.shape[1]
    O = W3.shape[1]

    C = -(-E // (NW * CHUNK))
    C = -(-C // NBUF) * NBUF
    epad = NW * C * CHUNK - E
    slots = (N + CHUNK) // CHUNK * CHUNK
    acc_rows = slots
    deg_slots = slots
    rows = 1000

    padn = jnp.full((epad,), N, jnp.int32)
    pad0 = jnp.zeros((epad,), jnp.int32)
    src_p = jnp.concatenate([edge_index[0], padn]).reshape(NW, C, CHUNK)
    dst_p = jnp.concatenate([edge_index[1], padn]).reshape(NW, C, CHUNK)
    src_pa = jnp.concatenate([edge_index[0], pad0]).reshape(NW, C, CHUNK)

    C2 = -(-E // (NS * CHUNK))
    C2 = -(-C2 // NBUF) * NBUF
    epad2 = NS * C2 * CHUNK - E
    src_p2 = jnp.concatenate(
        [edge_index[0], jnp.zeros((epad2,), jnp.int32)]).reshape(NS, C2, CHUNK)
    dst_p2 = jnp.concatenate(
        [edge_index[1], jnp.full((epad2,), N, jnp.int32)]).reshape(NS, C2, CHUNK)
    src_pc = jnp.stack([src_p2, src_p2 + slots])

    Dh = H // 2
    z_deg = jnp.zeros((deg_slots,), jnp.float32)
    z_acc_h = jnp.zeros((acc_rows, Dh), jnp.float32)
    z_acc_o = jnp.zeros((acc_rows, O), jnp.float32)

    pad_h = jnp.zeros((acc_rows - N, H // 2), jnp.float32)

    def split_t(t):
        return jnp.concatenate([t[:, :Dh], pad_h, t[:, Dh:], pad_h])

    deg_k = _make_deg_kernel(C, deg_slots)
    agg_h = _make_agg_cols_kernel(C2, Dh, N, acc_rows)
    agg_o = _make_agg_kernel(C, O, N, acc_rows)

    degp = deg_k(src_p, dst_p, z_deg)
    norms = _norm_call(degp.reshape(NW, 2 * deg_slots))
    norms = norms.reshape(2, deg_slots)
    n_out = norms[0, :N].reshape(N, 1)
    n_in = norms[1, :N].reshape(N, 1)

    t1 = _l1_call(features, n_out, W1, rows)
    p1 = agg_h(split_t(t1), src_pc, dst_p2, z_acc_h)
    t2 = _mid_call(p1, t1, n_in, n_out, b1.reshape(1, H), W2, rows)
    p2 = agg_h(split_t(t2), src_pc, dst_p2, z_acc_h)
    t3 = _mid_call(p2, t2, n_in, n_out, b2.reshape(1, H), W3, rows)
    p3 = agg_o(t3, src_pa, dst_p, z_acc_o)
    return _fin_call(p3, t3, n_in, b3.reshape(1, O), rows)


# speedup vs baseline: 1.2538x; 1.2538x over previous
import functools

import jax
import jax.numpy as jnp
from jax import lax
from jax.experimental import pallas as pl
from jax.experimental.pallas import tpu as pltpu
from jax.experimental.pallas import tpu_sc as plsc

NC = 2
NS = 16
NW = NC * NS
CHUNK = 128


def _make_deg_kernel(C, deg_slots):
    mesh = plsc.VectorSubcoreMesh(core_axis_name="c", subcore_axis_name="s")

    @functools.partial(
        pl.kernel,
        out_type=jax.ShapeDtypeStruct((NW * 2 * deg_slots,), jnp.float32),
        mesh=mesh,
        scratch_types=[
            pltpu.VMEM((C, CHUNK), jnp.int32),
            pltpu.VMEM((C, CHUNK), jnp.int32),
            pltpu.VMEM((deg_slots,), jnp.float32),
            pltpu.VMEM((deg_slots,), jnp.float32),
        ],
        compiler_params=pltpu.CompilerParams(needs_layout_passes=False),
    )
    def deg_kernel(src_hbm, dst_hbm, zeros_hbm, out_hbm,
                   src_v, dst_v, dego_v, degi_v):
        wid = lax.axis_index("s") * NC + lax.axis_index("c")
        pltpu.sync_copy(src_hbm.at[wid], src_v)
        pltpu.sync_copy(dst_hbm.at[wid], dst_v)
        pltpu.sync_copy(zeros_hbm, dego_v)
        pltpu.sync_copy(zeros_hbm, degi_v)
        ones = jnp.ones((16,), jnp.float32)

        @pl.loop(0, C)
        def _(r):
            for q in range(CHUNK // 16):
                s16 = src_v[r, pl.ds(q * 16, 16)]
                d16 = dst_v[r, pl.ds(q * 16, 16)]
                plsc.addupdate_scatter(dego_v, [s16], ones)
                plsc.addupdate_scatter(degi_v, [d16], ones)

        base = wid * 2 * deg_slots
        pltpu.sync_copy(dego_v, out_hbm.at[pl.ds(base, deg_slots)])
        pltpu.sync_copy(degi_v, out_hbm.at[pl.ds(base + deg_slots, deg_slots)])

    return deg_kernel


NBUF = 2


def _make_agg_cols_kernel(C, Dh, N, acc_rows):
    mesh = plsc.VectorSubcoreMesh(core_axis_name="c", subcore_axis_name="s")
    rows_pt = (N // NS) // 8 * 8
    rows_rem = N - NS * rows_pt
    zrows_pt = acc_rows // NS

    @functools.partial(
        pl.kernel,
        out_type=jax.ShapeDtypeStruct((NC, N, Dh), jnp.float32),
        mesh=mesh,
        scratch_types=[
            pltpu.VMEM((C, CHUNK), jnp.int32),
            pltpu.VMEM((C, CHUNK), jnp.int32),
            [pltpu.VMEM((CHUNK, Dh), jnp.float32) for _ in range(NBUF)],
            [pltpu.SemaphoreType.DMA for _ in range(NBUF)],
            pltpu.VMEM_SHARED((acc_rows, Dh), jnp.float32),
        ],
        compiler_params=pltpu.CompilerParams(use_tc_tiling_on_sc=False),
    )
    def agg_kernel(t2_hbm, src_hbm, dst_hbm, zeros_hbm, out_hbm,
                   src_v, dst_v, bufs, sems, acc):
        c = lax.axis_index("c")
        s = lax.axis_index("s")
        pltpu.sync_copy(src_hbm.at[c, s], src_v)
        for k in range(NBUF):
            pltpu.async_copy(t2_hbm.at[src_v.at[k]], bufs[k], sems[k])
        pltpu.sync_copy(dst_hbm.at[s], dst_v)
        pltpu.sync_copy(zeros_hbm.at[pl.ds(s * zrows_pt, zrows_pt)],
                        acc.at[pl.ds(s * zrows_pt, zrows_pt)])
        plsc.subcore_barrier()

        @pl.loop(0, C - NBUF, step=NBUF)
        def _(j):
            for k in range(NBUF):
                pltpu.make_async_copy(
                    t2_hbm.at[src_v.at[0]], bufs[k], sems[k]).wait()
                pltpu.sync_copy(bufs[k], acc.at[dst_v.at[j + k]], add=True)
                pltpu.async_copy(
                    t2_hbm.at[src_v.at[j + k + NBUF]], bufs[k], sems[k])

        for k in range(NBUF):
            pltpu.make_async_copy(
                t2_hbm.at[src_v.at[0]], bufs[k], sems[k]).wait()
            pltpu.sync_copy(bufs[k], acc.at[dst_v.at[C - NBUF + k]], add=True)

        plsc.subcore_barrier()
        pltpu.sync_copy(acc.at[pl.ds(s * rows_pt, rows_pt)],
                        out_hbm.at[c, pl.ds(s * rows_pt, rows_pt)])
        if rows_rem:
            @pl.when(s == 0)
            def _():
                pltpu.sync_copy(acc.at[pl.ds(NS * rows_pt, rows_rem)],
                                out_hbm.at[c, pl.ds(NS * rows_pt, rows_rem)])

    return agg_kernel


def _make_agg_kernel(C, D, N, acc_rows):
    mesh = plsc.VectorSubcoreMesh(core_axis_name="c", subcore_axis_name="s")
    rows_pt = (N // NS) // 8 * 8
    rows_rem = N - NS * rows_pt
    zrows_pt = acc_rows // NS

    @functools.partial(
        pl.kernel,
        out_type=jax.ShapeDtypeStruct((NC, N, D), jnp.float32),
        mesh=mesh,
        scratch_types=[
            pltpu.VMEM((C, CHUNK), jnp.int32),
            pltpu.VMEM((C, CHUNK), jnp.int32),
            [pltpu.VMEM((CHUNK, D), jnp.float32) for _ in range(NBUF)],
            [pltpu.SemaphoreType.DMA for _ in range(NBUF)],
            pltpu.VMEM_SHARED((acc_rows, D), jnp.float32),
        ],
        compiler_params=pltpu.CompilerParams(use_tc_tiling_on_sc=False),
    )
    def agg_kernel(t_hbm, src_hbm, dst_hbm, zeros_hbm, out_hbm,
                   src_v, dst_v, bufs, sems, acc):
        c = lax.axis_index("c")
        s = lax.axis_index("s")
        wid = s * NC + c
        pltpu.sync_copy(src_hbm.at[wid], src_v)
        for k in range(NBUF):
            pltpu.async_copy(t_hbm.at[src_v.at[k]], bufs[k], sems[k])
        pltpu.sync_copy(dst_hbm.at[wid], dst_v)
        pltpu.sync_copy(zeros_hbm.at[pl.ds(s * zrows_pt, zrows_pt)],
                        acc.at[pl.ds(s * zrows_pt, zrows_pt)])
        plsc.subcore_barrier()

        @pl.loop(0, C - NBUF, step=NBUF)
        def _(j):
            for k in range(NBUF):
                pltpu.make_async_copy(
                    t_hbm.at[src_v.at[0]], bufs[k], sems[k]).wait()
                pltpu.sync_copy(bufs[k], acc.at[dst_v.at[j + k]], add=True)
                pltpu.async_copy(
                    t_hbm.at[src_v.at[j + k + NBUF]], bufs[k], sems[k])

        for k in range(NBUF):
            pltpu.make_async_copy(
                t_hbm.at[src_v.at[0]], bufs[k], sems[k]).wait()
            pltpu.sync_copy(bufs[k], acc.at[dst_v.at[C - NBUF + k]], add=True)

        plsc.subcore_barrier()
        pltpu.sync_copy(acc.at[pl.ds(s * rows_pt, rows_pt)],
                        out_hbm.at[c, pl.ds(s * rows_pt, rows_pt)])
        if rows_rem:
            @pl.when(s == 0)
            def _():
                pltpu.sync_copy(acc.at[pl.ds(NS * rows_pt, rows_rem)],
                                out_hbm.at[c, pl.ds(NS * rows_pt, rows_rem)])

    return agg_kernel


def _norm_body(degp_ref, out_ref):
    d = jnp.sum(degp_ref[...], axis=0, keepdims=True) + 1.0
    out_ref[...] = lax.rsqrt(d)


def _l1_body(f_ref, no_ref, w_ref, out_ref):
    x = f_ref[...] * no_ref[...]
    out_ref[...] = jnp.dot(x, w_ref[...], preferred_element_type=jnp.float32)


def _mid_body(p_ref, t_ref, ni_ref, no_ref, b_ref, w_ref, out_ref):
    agg = jnp.concatenate([p_ref[0], p_ref[1]], axis=-1)
    x = (agg + t_ref[...]) * ni_ref[...] + b_ref[...]
    x = jnp.maximum(x, 0.0) * no_ref[...]
    out_ref[...] = jnp.dot(x, w_ref[...], preferred_element_type=jnp.float32)


def _fin_body(p_ref, t_ref, ni_ref, b_ref, out_ref):
    x = (p_ref[0] + p_ref[1] + t_ref[...]) * ni_ref[...] + b_ref[...]
    out_ref[...] = jax.nn.sigmoid(x) + 1e-8


def _norm_call(degp):
    nw, m = degp.shape
    return pl.pallas_call(
        _norm_body,
        out_shape=jax.ShapeDtypeStruct((1, m), jnp.float32),
    )(degp)


def _l1_call(features, n_out, W, rows):
    n, f = features.shape
    h = W.shape[1]
    grid = (n // rows,)
    return pl.pallas_call(
        _l1_body,
        grid=grid,
        in_specs=[
            pl.BlockSpec((rows, f), lambda i: (i, 0)),
            pl.BlockSpec((rows, 1), lambda i: (i, 0)),
            pl.BlockSpec((f, h), lambda i: (0, 0)),
        ],
        out_specs=pl.BlockSpec((rows, h), lambda i: (i, 0)),
        out_shape=jax.ShapeDtypeStruct((n, h), jnp.float32),
    )(features, n_out, W)


def _mid_call(p, t, n_in, n_out, b, W, rows):
    n, d = t.shape
    do = W.shape[1]
    grid = (n // rows,)
    return pl.pallas_call(
        _mid_body,
        grid=grid,
        in_specs=[
            pl.BlockSpec((NC, rows, d // 2), lambda i: (0, i, 0)),
            pl.BlockSpec((rows, d), lambda i: (i, 0)),
            pl.BlockSpec((rows, 1), lambda i: (i, 0)),
            pl.BlockSpec((rows, 1), lambda i: (i, 0)),
            pl.BlockSpec((1, d), lambda i: (0, 0)),
            pl.BlockSpec((d, do), lambda i: (0, 0)),
        ],
        out_specs=pl.BlockSpec((rows, do), lambda i: (i, 0)),
        out_shape=jax.ShapeDtypeStruct((n, do), jnp.float32),
    )(p, t, n_in, n_out, b, W)


def _fin_call(p, t, n_in, b, rows):
    n, d = t.shape
    grid = (n // rows,)
    return pl.pallas_call(
        _fin_body,
        grid=grid,
        in_specs=[
            pl.BlockSpec((NC, rows, d), lambda i: (0, i, 0)),
            pl.BlockSpec((rows, d), lambda i: (i, 0)),
            pl.BlockSpec((rows, 1), lambda i: (i, 0)),
            pl.BlockSpec((1, d), lambda i: (0, 0)),
        ],
        out_specs=pl.BlockSpec((rows, d), lambda i: (i, 0)),
        out_shape=jax.ShapeDtypeStruct((n, d), jnp.float32),
    )(p, t, n_in, b)


@jax.jit
def kernel(features, edge_index, W1, b1, W2, b2, W3, b3):
    N, F = features.shape
    E = edge_index.shape[1]
    H = W1.shape[1]
    O = W3.shape[1]

    C = -(-E // (NW * CHUNK))
    C = -(-C // NBUF) * NBUF
    epad = NW * C * CHUNK - E
    slots = (N + CHUNK) // CHUNK * CHUNK
    acc_rows = slots
    deg_slots = slots
    rows = 1000

    padn = jnp.full((epad,), N, jnp.int32)
    pad0 = jnp.zeros((epad,), jnp.int32)
    src_p = jnp.concatenate([edge_index[0], padn]).reshape(NW, C, CHUNK)
    dst_p = jnp.concatenate([edge_index[1], padn]).reshape(NW, C, CHUNK)
    src_pa = jnp.concatenate([edge_index[0], pad0]).reshape(NW, C, CHUNK)

    C2 = -(-E // (NS * CHUNK))
    C2 = -(-C2 // NBUF) * NBUF
    epad2 = NS * C2 * CHUNK - E
    src_p2 = jnp.concatenate(
        [edge_index[0], jnp.zeros((epad2,), jnp.int32)]).reshape(NS, C2, CHUNK)
    dst_p2 = jnp.concatenate(
        [edge_index[1], jnp.full((epad2,), N, jnp.int32)]).reshape(NS, C2, CHUNK)
    src_pc = jnp.stack([src_p2, src_p2 + slots])

    Dh = H // 2
    z_deg = jnp.zeros((deg_slots,), jnp.float32)
    z_acc_h = jnp.zeros((acc_rows, Dh), jnp.float32)
    z_acc_o = jnp.zeros((acc_rows, O), jnp.float32)

    pad_h = jnp.zeros((acc_rows - N, H // 2), jnp.float32)

    def split_t(t):
        return jnp.concatenate([t[:, :Dh], pad_h, t[:, Dh:], pad_h])

    deg_k = _make_deg_kernel(C, deg_slots)
    agg_h = _make_agg_cols_kernel(C2, Dh, N, acc_rows)
    agg_o = _make_agg_kernel(C, O, N, acc_rows)

    degp = deg_k(src_p, dst_p, z_deg)
    norms = _norm_call(degp.reshape(NW, 2 * deg_slots))
    norms = norms.reshape(2, deg_slots)
    n_out = norms[0, :N].reshape(N, 1)
    n_in = norms[1, :N].reshape(N, 1)

    t1 = _l1_call(features, n_out, W1, rows)
    p1 = agg_h(split_t(t1), src_pc, dst_p2, z_acc_h)
    t2 = _mid_call(p1, t1, n_in, n_out, b1.reshape(1, H), W2, rows)
    p2 = agg_h(split_t(t2), src_pc, dst_p2, z_acc_h)
    t3 = _mid_call(p2, t2, n_in, n_out, b2.reshape(1, H), W3, rows)
    p3 = agg_o(t3, src_pa, dst_p, z_acc_o)
    return _fin_call(p3, t3, n_in, b3.reshape(1, O), rows)


# speedup vs baseline: 1.2869x; 1.0264x over previous
import functools

import jax
import jax.numpy as jnp
from jax import lax
from jax.experimental import pallas as pl
from jax.experimental.pallas import tpu as pltpu
from jax.experimental.pallas import tpu_sc as plsc

NC = 2
NS = 16
NW = NC * NS
CHUNK = 128


def _make_deg_kernel(C, deg_slots):
    mesh = plsc.VectorSubcoreMesh(core_axis_name="c", subcore_axis_name="s")

    @functools.partial(
        pl.kernel,
        out_type=jax.ShapeDtypeStruct((NW * 2 * deg_slots,), jnp.float32),
        mesh=mesh,
        scratch_types=[
            pltpu.VMEM((C, CHUNK), jnp.int32),
            pltpu.VMEM((C, CHUNK), jnp.int32),
            pltpu.VMEM((deg_slots,), jnp.float32),
            pltpu.VMEM((deg_slots,), jnp.float32),
        ],
        compiler_params=pltpu.CompilerParams(needs_layout_passes=False),
    )
    def deg_kernel(src_hbm, dst_hbm, zeros_hbm, out_hbm,
                   src_v, dst_v, dego_v, degi_v):
        wid = lax.axis_index("s") * NC + lax.axis_index("c")
        pltpu.sync_copy(src_hbm.at[wid], src_v)
        pltpu.sync_copy(dst_hbm.at[wid], dst_v)
        pltpu.sync_copy(zeros_hbm, dego_v)
        pltpu.sync_copy(zeros_hbm, degi_v)
        ones = jnp.ones((16,), jnp.float32)

        @pl.loop(0, C)
        def _(r):
            for q in range(CHUNK // 16):
                s16 = src_v[r, pl.ds(q * 16, 16)]
                d16 = dst_v[r, pl.ds(q * 16, 16)]
                plsc.addupdate_scatter(dego_v, [s16], ones)
                plsc.addupdate_scatter(degi_v, [d16], ones)

        base = wid * 2 * deg_slots
        pltpu.sync_copy(dego_v, out_hbm.at[pl.ds(base, deg_slots)])
        pltpu.sync_copy(degi_v, out_hbm.at[pl.ds(base + deg_slots, deg_slots)])

    return deg_kernel


NBUF = 2


def _make_agg_cols_kernel(C, Dh, N, acc_rows):
    mesh = plsc.VectorSubcoreMesh(core_axis_name="c", subcore_axis_name="s")
    rows_pt = (N // NS) // 8 * 8
    rows_rem = N - NS * rows_pt
    zrows_pt = acc_rows // NS

    @functools.partial(
        pl.kernel,
        out_type=jax.ShapeDtypeStruct((NC, N, Dh), jnp.float32),
        mesh=mesh,
        scratch_types=[
            pltpu.VMEM((C, CHUNK), jnp.int32),
            pltpu.VMEM((C, CHUNK), jnp.int32),
            [pltpu.VMEM((CHUNK, Dh), jnp.float32) for _ in range(NBUF)],
            [pltpu.SemaphoreType.DMA for _ in range(NBUF)],
            pltpu.VMEM_SHARED((acc_rows, Dh), jnp.float32),
        ],
        compiler_params=pltpu.CompilerParams(use_tc_tiling_on_sc=False),
    )
    def agg_kernel(t2_hbm, src_hbm, dst_hbm, zeros_hbm, out_hbm,
                   src_v, dst_v, bufs, sems, acc):
        c = lax.axis_index("c")
        s = lax.axis_index("s")
        pltpu.sync_copy(src_hbm.at[c, s], src_v)
        for k in range(NBUF):
            pltpu.async_copy(t2_hbm.at[src_v.at[k]], bufs[k], sems[k])
        pltpu.sync_copy(dst_hbm.at[s], dst_v)
        pltpu.sync_copy(zeros_hbm.at[pl.ds(s * zrows_pt, zrows_pt)],
                        acc.at[pl.ds(s * zrows_pt, zrows_pt)])
        plsc.subcore_barrier()

        @pl.loop(0, C - NBUF, step=NBUF)
        def _(j):
            for k in range(NBUF):
                pltpu.make_async_copy(
                    t2_hbm.at[src_v.at[0]], bufs[k], sems[k]).wait()
                pltpu.sync_copy(bufs[k], acc.at[dst_v.at[j + k]], add=True)
                pltpu.async_copy(
                    t2_hbm.at[src_v.at[j + k + NBUF]], bufs[k], sems[k])

        for k in range(NBUF):
            pltpu.make_async_copy(
                t2_hbm.at[src_v.at[0]], bufs[k], sems[k]).wait()
            pltpu.sync_copy(bufs[k], acc.at[dst_v.at[C - NBUF + k]], add=True)

        plsc.subcore_barrier()
        pltpu.sync_copy(acc.at[pl.ds(s * rows_pt, rows_pt)],
                        out_hbm.at[c, pl.ds(s * rows_pt, rows_pt)])
        if rows_rem:
            @pl.when(s == 0)
            def _():
                pltpu.sync_copy(acc.at[pl.ds(NS * rows_pt, rows_rem)],
                                out_hbm.at[c, pl.ds(NS * rows_pt, rows_rem)])

    return agg_kernel


def _make_agg_kernel(C, D, N, acc_rows):
    mesh = plsc.VectorSubcoreMesh(core_axis_name="c", subcore_axis_name="s")
    rows_pt = (N // NS) // 8 * 8
    rows_rem = N - NS * rows_pt
    zrows_pt = acc_rows // NS

    @functools.partial(
        pl.kernel,
        out_type=jax.ShapeDtypeStruct((NC, N, D), jnp.float32),
        mesh=mesh,
        scratch_types=[
            pltpu.VMEM((C, CHUNK), jnp.int32),
            pltpu.VMEM((C, CHUNK), jnp.int32),
            [pltpu.VMEM((CHUNK, D), jnp.float32) for _ in range(NBUF)],
            [pltpu.SemaphoreType.DMA for _ in range(NBUF)],
            pltpu.VMEM_SHARED((acc_rows, D), jnp.float32),
        ],
        compiler_params=pltpu.CompilerParams(use_tc_tiling_on_sc=False),
    )
    def agg_kernel(t_hbm, src_hbm, dst_hbm, zeros_hbm, out_hbm,
                   src_v, dst_v, bufs, sems, acc):
        c = lax.axis_index("c")
        s = lax.axis_index("s")
        wid = s * NC + c
        pltpu.sync_copy(src_hbm.at[wid], src_v)
        for k in range(NBUF):
            pltpu.async_copy(t_hbm.at[src_v.at[k]], bufs[k], sems[k])
        pltpu.sync_copy(dst_hbm.at[wid], dst_v)
        pltpu.sync_copy(zeros_hbm.at[pl.ds(s * zrows_pt, zrows_pt)],
                        acc.at[pl.ds(s * zrows_pt, zrows_pt)])
        plsc.subcore_barrier()

        @pl.loop(0, C - NBUF, step=NBUF)
        def _(j):
            for k in range(NBUF):
                pltpu.make_async_copy(
                    t_hbm.at[src_v.at[0]], bufs[k], sems[k]).wait()
                pltpu.sync_copy(bufs[k], acc.at[dst_v.at[j + k]], add=True)
                pltpu.async_copy(
                    t_hbm.at[src_v.at[j + k + NBUF]], bufs[k], sems[k])

        for k in range(NBUF):
            pltpu.make_async_copy(
                t_hbm.at[src_v.at[0]], bufs[k], sems[k]).wait()
            pltpu.sync_copy(bufs[k], acc.at[dst_v.at[C - NBUF + k]], add=True)

        plsc.subcore_barrier()
        pltpu.sync_copy(acc.at[pl.ds(s * rows_pt, rows_pt)],
                        out_hbm.at[c, pl.ds(s * rows_pt, rows_pt)])
        if rows_rem:
            @pl.when(s == 0)
            def _():
                pltpu.sync_copy(acc.at[pl.ds(NS * rows_pt, rows_rem)],
                                out_hbm.at[c, pl.ds(NS * rows_pt, rows_rem)])

    return agg_kernel


def _norm_body(degp_ref, out_ref):
    d = jnp.sum(degp_ref[...], axis=0, keepdims=True) + 1.0
    out_ref[...] = lax.rsqrt(d)


def _l1_body(f_ref, no_ref, w_ref, out_ref, tab_ref):
    x = f_ref[...] * no_ref[...]
    t = jnp.dot(x, w_ref[...], preferred_element_type=jnp.float32)
    out_ref[...] = t
    h = t.shape[1]
    tab_ref[0] = t[:, : h // 2]
    tab_ref[1] = t[:, h // 2:]


def _mid_body(p_ref, t_ref, ni_ref, no_ref, b_ref, w_ref, out_ref,
              tab_ref=None):
    agg = jnp.concatenate([p_ref[0], p_ref[1]], axis=-1)
    x = (agg + t_ref[...]) * ni_ref[...] + b_ref[...]
    x = jnp.maximum(x, 0.0) * no_ref[...]
    t = jnp.dot(x, w_ref[...], preferred_element_type=jnp.float32)
    out_ref[...] = t
    if tab_ref is not None:
        h = t.shape[1]
        tab_ref[0] = t[:, : h // 2]
        tab_ref[1] = t[:, h // 2:]


def _fin_body(p_ref, t_ref, ni_ref, b_ref, out_ref):
    x = (p_ref[0] + p_ref[1] + t_ref[...]) * ni_ref[...] + b_ref[...]
    out_ref[...] = jax.nn.sigmoid(x) + 1e-8


def _norm_call(degp):
    nw, m = degp.shape
    return pl.pallas_call(
        _norm_body,
        out_shape=jax.ShapeDtypeStruct((1, m), jnp.float32),
    )(degp)


def _l1_call(features, n_out, W, rows, slots):
    n, f = features.shape
    h = W.shape[1]
    grid = (n // rows,)
    return pl.pallas_call(
        _l1_body,
        grid=grid,
        in_specs=[
            pl.BlockSpec((rows, f), lambda i: (i, 0)),
            pl.BlockSpec((rows, 1), lambda i: (i, 0)),
            pl.BlockSpec((f, h), lambda i: (0, 0)),
        ],
        out_specs=[
            pl.BlockSpec((rows, h), lambda i: (i, 0)),
            pl.BlockSpec((2, rows, h // 2), lambda i: (0, i, 0)),
        ],
        out_shape=[
            jax.ShapeDtypeStruct((n, h), jnp.float32),
            jax.ShapeDtypeStruct((2, slots, h // 2), jnp.float32),
        ],
    )(features, n_out, W)


def _mid_call(p, t, n_in, n_out, b, W, rows, slots=None):
    n, d = t.shape
    do = W.shape[1]
    grid = (n // rows,)
    out_specs = [pl.BlockSpec((rows, do), lambda i: (i, 0))]
    out_shape = [jax.ShapeDtypeStruct((n, do), jnp.float32)]
    if slots is not None:
        out_specs.append(pl.BlockSpec((2, rows, do // 2), lambda i: (0, i, 0)))
        out_shape.append(jax.ShapeDtypeStruct((2, slots, do // 2), jnp.float32))
    res = pl.pallas_call(
        _mid_body,
        grid=grid,
        in_specs=[
            pl.BlockSpec((NC, rows, d // 2), lambda i: (0, i, 0)),
            pl.BlockSpec((rows, d), lambda i: (i, 0)),
            pl.BlockSpec((rows, 1), lambda i: (i, 0)),
            pl.BlockSpec((rows, 1), lambda i: (i, 0)),
            pl.BlockSpec((1, d), lambda i: (0, 0)),
            pl.BlockSpec((d, do), lambda i: (0, 0)),
        ],
        out_specs=out_specs,
        out_shape=out_shape,
    )(p, t, n_in, n_out, b, W)
    return res if slots is not None else res[0]


def _fin_call(p, t, n_in, b, rows):
    n, d = t.shape
    grid = (n // rows,)
    return pl.pallas_call(
        _fin_body,
        grid=grid,
        in_specs=[
            pl.BlockSpec((NC, rows, d), lambda i: (0, i, 0)),
            pl.BlockSpec((rows, d), lambda i: (i, 0)),
            pl.BlockSpec((rows, 1), lambda i: (i, 0)),
            pl.BlockSpec((1, d), lambda i: (0, 0)),
        ],
        out_specs=pl.BlockSpec((rows, d), lambda i: (i, 0)),
        out_shape=jax.ShapeDtypeStruct((n, d), jnp.float32),
    )(p, t, n_in, b)


@jax.jit
def kernel(features, edge_index, W1, b1, W2, b2, W3, b3):
    N, F = features.shape
    E = edge_index.shape[1]
    H = W1.shape[1]
    O = W3.shape[1]

    C = -(-E // (NW * CHUNK))
    C = -(-C // NBUF) * NBUF
    epad = NW * C * CHUNK - E
    slots = (N + CHUNK) // CHUNK * CHUNK
    acc_rows = slots
    deg_slots = slots
    rows = 1000

    padn = jnp.full((epad,), N, jnp.int32)
    pad0 = jnp.zeros((epad,), jnp.int32)
    src_p = jnp.concatenate([edge_index[0], padn]).reshape(NW, C, CHUNK)
    dst_p = jnp.concatenate([edge_index[1], padn]).reshape(NW, C, CHUNK)
    src_pa = jnp.concatenate([edge_index[0], pad0]).reshape(NW, C, CHUNK)

    C2 = -(-E // (NS * CHUNK))
    C2 = -(-C2 // NBUF) * NBUF
    epad2 = NS * C2 * CHUNK - E
    src_p2 = jnp.concatenate(
        [edge_index[0], jnp.zeros((epad2,), jnp.int32)]).reshape(NS, C2, CHUNK)
    dst_p2 = jnp.concatenate(
        [edge_index[1], jnp.full((epad2,), N, jnp.int32)]).reshape(NS, C2, CHUNK)
    src_pc = jnp.stack([src_p2, src_p2 + slots])

    Dh = H // 2
    z_deg = jnp.zeros((deg_slots,), jnp.float32)
    z_acc_h = jnp.zeros((acc_rows, Dh), jnp.float32)
    z_acc_o = jnp.zeros((acc_rows, O), jnp.float32)

    deg_k = _make_deg_kernel(C, deg_slots)
    agg_h = _make_agg_cols_kernel(C2, Dh, N, acc_rows)
    agg_o = _make_agg_kernel(C, O, N, acc_rows)

    degp = deg_k(src_p, dst_p, z_deg)
    norms = _norm_call(degp.reshape(NW, 2 * deg_slots))
    norms = norms.reshape(2, deg_slots)
    n_out = norms[0, :N].reshape(N, 1)
    n_in = norms[1, :N].reshape(N, 1)

    t1, tb1 = _l1_call(features, n_out, W1, rows, slots)
    p1 = agg_h(tb1.reshape(2 * slots, Dh), src_pc, dst_p2, z_acc_h)
    t2, tb2 = _mid_call(p1, t1, n_in, n_out, b1.reshape(1, H), W2, rows, slots)
    p2 = agg_h(tb2.reshape(2 * slots, Dh), src_pc, dst_p2, z_acc_h)
    t3 = _mid_call(p2, t2, n_in, n_out, b2.reshape(1, H), W3, rows)
    p3 = agg_o(t3, src_pa, dst_p, z_acc_o)
    return _fin_call(p3, t3, n_in, b3.reshape(1, O), rows)


# speedup vs baseline: 1.3000x; 1.0102x over previous
import functools

import jax
import jax.numpy as jnp
from jax import lax
from jax.experimental import pallas as pl
from jax.experimental.pallas import tpu as pltpu
from jax.experimental.pallas import tpu_sc as plsc

NC = 2
NS = 16
NW = NC * NS
CHUNK = 128


def _make_deg_kernel(C, deg_slots):
    mesh = plsc.VectorSubcoreMesh(core_axis_name="c", subcore_axis_name="s")

    @functools.partial(
        pl.kernel,
        out_type=jax.ShapeDtypeStruct((NW * 2 * deg_slots,), jnp.float32),
        mesh=mesh,
        scratch_types=[
            pltpu.VMEM((C, CHUNK), jnp.int32),
            pltpu.VMEM((C, CHUNK), jnp.int32),
            pltpu.VMEM((deg_slots,), jnp.float32),
            pltpu.VMEM((deg_slots,), jnp.float32),
        ],
        compiler_params=pltpu.CompilerParams(needs_layout_passes=False),
    )
    def deg_kernel(src_hbm, dst_hbm, zeros_hbm, out_hbm,
                   src_v, dst_v, dego_v, degi_v):
        wid = lax.axis_index("s") * NC + lax.axis_index("c")
        pltpu.sync_copy(src_hbm.at[wid], src_v)
        pltpu.sync_copy(dst_hbm.at[wid], dst_v)
        pltpu.sync_copy(zeros_hbm, dego_v)
        pltpu.sync_copy(zeros_hbm, degi_v)
        ones = jnp.ones((16,), jnp.float32)

        @pl.loop(0, C)
        def _(r):
            for q in range(CHUNK // 16):
                s16 = src_v[r, pl.ds(q * 16, 16)]
                d16 = dst_v[r, pl.ds(q * 16, 16)]
                plsc.addupdate_scatter(dego_v, [s16], ones)
                plsc.addupdate_scatter(degi_v, [d16], ones)

        base = wid * 2 * deg_slots
        pltpu.sync_copy(dego_v, out_hbm.at[pl.ds(base, deg_slots)])
        pltpu.sync_copy(degi_v, out_hbm.at[pl.ds(base + deg_slots, deg_slots)])

    return deg_kernel


NBUF = 2


def _make_agg_cols_kernel(C, Dh, N, acc_rows):
    mesh = plsc.VectorSubcoreMesh(core_axis_name="c", subcore_axis_name="s")
    rows_pt = (N // NS) // 8 * 8
    rows_rem = N - NS * rows_pt
    zrows_pt = acc_rows // NS

    @functools.partial(
        pl.kernel,
        out_type=jax.ShapeDtypeStruct((NC, N, Dh), jnp.float32),
        mesh=mesh,
        scratch_types=[
            pltpu.VMEM((C, CHUNK), jnp.int32),
            pltpu.VMEM((C, CHUNK), jnp.int32),
            [pltpu.VMEM((CHUNK, Dh), jnp.float32) for _ in range(NBUF)],
            [pltpu.SemaphoreType.DMA for _ in range(NBUF)],
            pltpu.VMEM_SHARED((acc_rows, Dh), jnp.float32),
        ],
        compiler_params=pltpu.CompilerParams(use_tc_tiling_on_sc=False),
    )
    def agg_kernel(t2_hbm, src_hbm, dst_hbm, zeros_hbm, out_hbm,
                   src_v, dst_v, bufs, sems, acc):
        c = lax.axis_index("c")
        s = lax.axis_index("s")
        pltpu.sync_copy(src_hbm.at[c, s], src_v)
        for k in range(NBUF):
            pltpu.async_copy(t2_hbm.at[src_v.at[k]], bufs[k], sems[k])
        pltpu.sync_copy(dst_hbm.at[s], dst_v)
        pltpu.sync_copy(zeros_hbm.at[pl.ds(s * zrows_pt, zrows_pt)],
                        acc.at[pl.ds(s * zrows_pt, zrows_pt)])
        plsc.subcore_barrier()

        @pl.loop(0, C - NBUF, step=NBUF)
        def _(j):
            for k in range(NBUF):
                pltpu.make_async_copy(
                    t2_hbm.at[src_v.at[0]], bufs[k], sems[k]).wait()
                pltpu.sync_copy(bufs[k], acc.at[dst_v.at[j + k]], add=True)
                pltpu.async_copy(
                    t2_hbm.at[src_v.at[j + k + NBUF]], bufs[k], sems[k])

        for k in range(NBUF):
            pltpu.make_async_copy(
                t2_hbm.at[src_v.at[0]], bufs[k], sems[k]).wait()
            pltpu.sync_copy(bufs[k], acc.at[dst_v.at[C - NBUF + k]], add=True)

        plsc.subcore_barrier()
        pltpu.sync_copy(acc.at[pl.ds(s * rows_pt, rows_pt)],
                        out_hbm.at[c, pl.ds(s * rows_pt, rows_pt)])
        if rows_rem:
            @pl.when(s == 0)
            def _():
                pltpu.sync_copy(acc.at[pl.ds(NS * rows_pt, rows_rem)],
                                out_hbm.at[c, pl.ds(NS * rows_pt, rows_rem)])

    return agg_kernel


def _make_agg_kernel(C, D, N, acc_rows):
    mesh = plsc.VectorSubcoreMesh(core_axis_name="c", subcore_axis_name="s")
    rows_pt = (N // NS) // 8 * 8
    rows_rem = N - NS * rows_pt
    zrows_pt = acc_rows // NS

    @functools.partial(
        pl.kernel,
        out_type=jax.ShapeDtypeStruct((NC, N, D), jnp.float32),
        mesh=mesh,
        scratch_types=[
            pltpu.VMEM((C, CHUNK), jnp.int32),
            pltpu.VMEM((C, CHUNK), jnp.int32),
            [pltpu.VMEM((CHUNK, D), jnp.float32) for _ in range(NBUF)],
            [pltpu.SemaphoreType.DMA for _ in range(NBUF)],
            pltpu.VMEM_SHARED((acc_rows, D), jnp.float32),
        ],
        compiler_params=pltpu.CompilerParams(use_tc_tiling_on_sc=False),
    )
    def agg_kernel(t_hbm, src_hbm, dst_hbm, zeros_hbm, out_hbm,
                   src_v, dst_v, bufs, sems, acc):
        c = lax.axis_index("c")
        s = lax.axis_index("s")
        wid = s * NC + c
        pltpu.sync_copy(src_hbm.at[wid], src_v)
        for k in range(NBUF):
            pltpu.async_copy(t_hbm.at[src_v.at[k]], bufs[k], sems[k])
        pltpu.sync_copy(dst_hbm.at[wid], dst_v)
        pltpu.sync_copy(zeros_hbm.at[pl.ds(s * zrows_pt, zrows_pt)],
                        acc.at[pl.ds(s * zrows_pt, zrows_pt)])
        plsc.subcore_barrier()

        @pl.loop(0, C - NBUF, step=NBUF)
        def _(j):
            for k in range(NBUF):
                pltpu.make_async_copy(
                    t_hbm.at[src_v.at[0]], bufs[k], sems[k]).wait()
                pltpu.sync_copy(bufs[k], acc.at[dst_v.at[j + k]], add=True)
                pltpu.async_copy(
                    t_hbm.at[src_v.at[j + k + NBUF]], bufs[k], sems[k])

        for k in range(NBUF):
            pltpu.make_async_copy(
                t_hbm.at[src_v.at[0]], bufs[k], sems[k]).wait()
            pltpu.sync_copy(bufs[k], acc.at[dst_v.at[C - NBUF + k]], add=True)

        plsc.subcore_barrier()
        pltpu.sync_copy(acc.at[pl.ds(s * rows_pt, rows_pt)],
                        out_hbm.at[c, pl.ds(s * rows_pt, rows_pt)])
        if rows_rem:
            @pl.when(s == 0)
            def _():
                pltpu.sync_copy(acc.at[pl.ds(NS * rows_pt, rows_rem)],
                                out_hbm.at[c, pl.ds(NS * rows_pt, rows_rem)])

    return agg_kernel


def _norm_body(degp_ref, out_ref):
    d = jnp.sum(degp_ref[...], axis=0, keepdims=True) + 1.0
    out_ref[...] = lax.rsqrt(d)


def _l1_body(f_ref, no_ref, w_ref, out_ref, tab_ref):
    x = f_ref[...] * no_ref[...]
    t = jnp.dot(x, w_ref[...], preferred_element_type=jnp.float32)
    out_ref[...] = t
    h = t.shape[1]
    tab_ref[0] = t[:, : h // 2]
    tab_ref[1] = t[:, h // 2:]


def _mid_body(p_ref, t_ref, ni_ref, no_ref, b_ref, w_ref, out_ref,
              tab_ref=None):
    agg = jnp.concatenate([p_ref[0], p_ref[1]], axis=-1)
    x = (agg + t_ref[...]) * ni_ref[...] + b_ref[...]
    x = jnp.maximum(x, 0.0) * no_ref[...]
    t = jnp.dot(x, w_ref[...], preferred_element_type=jnp.float32)
    out_ref[...] = t
    if tab_ref is not None:
        h = t.shape[1]
        tab_ref[0] = t[:, : h // 2]
        tab_ref[1] = t[:, h // 2:]


def _fin_body(p_ref, t_ref, ni_ref, b_ref, out_ref):
    x = (p_ref[0] + p_ref[1] + t_ref[...]) * ni_ref[...] + b_ref[...]
    out_ref[...] = jax.nn.sigmoid(x) + 1e-8


def _norm_call(degp):
    nw, m = degp.shape
    return pl.pallas_call(
        _norm_body,
        out_shape=jax.ShapeDtypeStruct((1, m), jnp.float32),
    )(degp)


def _l1_call(features, n_out, W, rows, slots):
    n, f = features.shape
    h = W.shape[1]
    grid = (n // rows,)
    return pl.pallas_call(
        _l1_body,
        grid=grid,
        in_specs=[
            pl.BlockSpec((rows, f), lambda i: (i, 0)),
            pl.BlockSpec((rows, 1), lambda i: (i, 0)),
            pl.BlockSpec((f, h), lambda i: (0, 0)),
        ],
        out_specs=[
            pl.BlockSpec((rows, h), lambda i: (i, 0)),
            pl.BlockSpec((2, rows, h // 2), lambda i: (0, i, 0)),
        ],
        out_shape=[
            jax.ShapeDtypeStruct((n, h), jnp.float32),
            jax.ShapeDtypeStruct((2, slots, h // 2), jnp.float32),
        ],
    )(features, n_out, W)


def _mid_call(p, t, n_in, n_out, b, W, rows, slots=None):
    n, d = t.shape
    do = W.shape[1]
    grid = (n // rows,)
    out_specs = [pl.BlockSpec((rows, do), lambda i: (i, 0))]
    out_shape = [jax.ShapeDtypeStruct((n, do), jnp.float32)]
    if slots is not None:
        out_specs.append(pl.BlockSpec((2, rows, do // 2), lambda i: (0, i, 0)))
        out_shape.append(jax.ShapeDtypeStruct((2, slots, do // 2), jnp.float32))
    res = pl.pallas_call(
        _mid_body,
        grid=grid,
        in_specs=[
            pl.BlockSpec((NC, rows, d // 2), lambda i: (0, i, 0)),
            pl.BlockSpec((rows, d), lambda i: (i, 0)),
            pl.BlockSpec((rows, 1), lambda i: (i, 0)),
            pl.BlockSpec((rows, 1), lambda i: (i, 0)),
            pl.BlockSpec((1, d), lambda i: (0, 0)),
            pl.BlockSpec((d, do), lambda i: (0, 0)),
        ],
        out_specs=out_specs,
        out_shape=out_shape,
    )(p, t, n_in, n_out, b, W)
    return res if slots is not None else res[0]


def _fin_call(p, t, n_in, b, rows):
    n, d = t.shape
    grid = (n // rows,)
    return pl.pallas_call(
        _fin_body,
        grid=grid,
        in_specs=[
            pl.BlockSpec((NC, rows, d), lambda i: (0, i, 0)),
            pl.BlockSpec((rows, d), lambda i: (i, 0)),
            pl.BlockSpec((rows, 1), lambda i: (i, 0)),
            pl.BlockSpec((1, d), lambda i: (0, 0)),
        ],
        out_specs=pl.BlockSpec((rows, d), lambda i: (i, 0)),
        out_shape=jax.ShapeDtypeStruct((n, d), jnp.float32),
    )(p, t, n_in, b)


@jax.jit
def kernel(features, edge_index, W1, b1, W2, b2, W3, b3):
    N, F = features.shape
    E = edge_index.shape[1]
    H = W1.shape[1]
    O = W3.shape[1]

    C = -(-E // (NW * CHUNK))
    C = -(-C // NBUF) * NBUF
    epad = NW * C * CHUNK - E
    slots = (N + CHUNK) // CHUNK * CHUNK
    acc_rows = slots
    deg_slots = slots
    rows = 2000

    padn = jnp.full((epad,), N, jnp.int32)
    pad0 = jnp.zeros((epad,), jnp.int32)
    src_p = jnp.concatenate([edge_index[0], padn]).reshape(NW, C, CHUNK)
    dst_p = jnp.concatenate([edge_index[1], padn]).reshape(NW, C, CHUNK)
    src_pa = jnp.concatenate([edge_index[0], pad0]).reshape(NW, C, CHUNK)

    C2 = -(-E // (NS * CHUNK))
    C2 = -(-C2 // NBUF) * NBUF
    epad2 = NS * C2 * CHUNK - E
    src_p2 = jnp.concatenate(
        [edge_index[0], jnp.zeros((epad2,), jnp.int32)]).reshape(NS, C2, CHUNK)
    dst_p2 = jnp.concatenate(
        [edge_index[1], jnp.full((epad2,), N, jnp.int32)]).reshape(NS, C2, CHUNK)
    src_pc = jnp.stack([src_p2, src_p2 + slots])

    Dh = H // 2
    z_deg = jnp.zeros((deg_slots,), jnp.float32)
    z_acc_h = jnp.zeros((acc_rows, Dh), jnp.float32)
    z_acc_o = jnp.zeros((acc_rows, O), jnp.float32)

    deg_k = _make_deg_kernel(C, deg_slots)
    agg_h = _make_agg_cols_kernel(C2, Dh, N, acc_rows)
    agg_o = _make_agg_kernel(C, O, N, acc_rows)

    degp = deg_k(src_p, dst_p, z_deg)
    norms = _norm_call(degp.reshape(NW, 2 * deg_slots))
    norms = norms.reshape(2, deg_slots)
    n_out = norms[0, :N].reshape(N, 1)
    n_in = norms[1, :N].reshape(N, 1)

    t1, tb1 = _l1_call(features, n_out, W1, rows, slots)
    p1 = agg_h(tb1.reshape(2 * slots, Dh), src_pc, dst_p2, z_acc_h)
    t2, tb2 = _mid_call(p1, t1, n_in, n_out, b1.reshape(1, H), W2, rows, slots)
    p2 = agg_h(tb2.reshape(2 * slots, Dh), src_pc, dst_p2, z_acc_h)
    t3 = _mid_call(p2, t2, n_in, n_out, b2.reshape(1, H), W3, rows)
    p3 = agg_o(t3, src_pa, dst_p, z_acc_o)
    return _fin_call(p3, t3, n_in, b3.reshape(1, O), rows)
